# baseline (speedup 1.0000x reference)
import jax
import jax.numpy as jnp
from jax import lax
from jax.experimental import pallas as pl
from jax.experimental.pallas import tpu as pltpu
from jax.experimental.pallas import tpu_sc as plsc

L = 2
D = 128
N = 10000
E = 320000

NC = 2
NS = 16
NW = NC * NS
G = 80
K = 5
EK = E // K
EPWC = EK // NW
NGC = EPWC // G
EPW = E // NW
NG = EPW // G
RC = 80
NRC = N // RC

_sc_cache = {}


def _sc_mesh():
    return plsc.VectorSubcoreMesh(
        core_axis_name="c", subcore_axis_name="s",
        num_cores=NC, num_subcores=NS)


def _sc_gather_body(x_hbm, row_hbm, col_hbm, xr_hbm, xc_hbm,
                    ridx_v, cidx_v, rbuf0, rbuf1, cbuf0, cbuf1,
                    sem_r0, sem_r1, sem_c0, sem_c1):
    wid = lax.axis_index("s") * NC + lax.axis_index("c")
    pltpu.sync_copy(row_hbm.at[wid], ridx_v)
    pltpu.sync_copy(col_hbm.at[wid], cidx_v)
    base = wid * EPWC
    bufs = ((rbuf0, cbuf0, sem_r0, sem_c0), (rbuf1, cbuf1, sem_r1, sem_c1))

    def issue(j, p):
        rb, cb, sr, sc = bufs[p]
        pltpu.async_copy(x_hbm.at[ridx_v.at[j]], rb, sr)
        pltpu.async_copy(x_hbm.at[cidx_v.at[j]], cb, sc)

    def drain_write(j, p):
        rb, cb, sr, sc = bufs[p]
        pltpu.make_async_copy(x_hbm.at[ridx_v.at[j]], rb, sr).wait()
        pltpu.sync_copy(rb, xr_hbm.at[pl.ds(base + j * G, G)])
        pltpu.make_async_copy(x_hbm.at[cidx_v.at[j]], cb, sc).wait()
        pltpu.sync_copy(cb, xc_hbm.at[pl.ds(base + j * G, G)])

    issue(0, 0)

    @pl.loop(0, (NGC - 1) // 2)
    def _(t):
        j = 2 * t
        issue(j + 1, 1)
        drain_write(j, 0)
        issue(j + 2, 0)
        drain_write(j + 1, 1)

    drain_write(NGC - 1, 0)


DH = D // 2


def _sc_gather(xb, rowk, colk):
    if "gather" not in _sc_cache:
        _sc_cache["gather"] = pl.kernel(
            _sc_gather_body,
            out_type=(
                jax.ShapeDtypeStruct((EK, DH), jnp.int32),
                jax.ShapeDtypeStruct((EK, DH), jnp.int32),
            ),
            mesh=_sc_mesh(),
            compiler_params=pltpu.CompilerParams(use_tc_tiling_on_sc=False),
            scratch_types=[
                pltpu.VMEM((NGC, G), jnp.int32),
                pltpu.VMEM((NGC, G), jnp.int32),
                pltpu.VMEM((G, DH), jnp.int32),
                pltpu.VMEM((G, DH), jnp.int32),
                pltpu.VMEM((G, DH), jnp.int32),
                pltpu.VMEM((G, DH), jnp.int32),
                pltpu.SemaphoreType.DMA,
                pltpu.SemaphoreType.DMA,
                pltpu.SemaphoreType.DMA,
                pltpu.SemaphoreType.DMA,
            ],
        )
    x32 = lax.bitcast_convert_type(xb.reshape(N, DH, 2), jnp.int32)
    xr32, xc32 = _sc_cache["gather"](x32, rowk, colk)
    xr = lax.bitcast_convert_type(xr32, jnp.bfloat16).reshape(EK, D)
    xc = lax.bitcast_convert_type(xc32, jnp.bfloat16).reshape(EK, D)
    return xr, xc


def _acc_zero(s, znd_hbm, acc_sh):
    for t in range((NRC + NS - 1) // NS):
        k = s + NS * t

        @pl.when(k < NRC)
        def _():
            pltpu.sync_copy(znd_hbm.at[pl.ds(k * RC, RC)],
                            acc_sh.at[pl.ds(k * RC, RC)])


def _acc_drain(c, s, acc_sh, acc_out):
    for t in range((NRC + NS - 1) // NS):
        k = s + NS * t

        @pl.when(k < NRC)
        def _():
            pltpu.sync_copy(acc_sh.at[pl.ds(k * RC, RC)],
                            acc_out.at[c, pl.ds(k * RC, RC)])


def _sc_scatter_data_body(ea_hbm, col_hbm, znd_hbm, acc_out,
                          cidx_v, ebuf0, ebuf1, sem0, sem1, acc_sh):
    c = lax.axis_index("c")
    s = lax.axis_index("s")
    wid = s * NC + c
    _acc_zero(s, znd_hbm, acc_sh)
    pltpu.sync_copy(col_hbm.at[wid], cidx_v)
    plsc.subcore_barrier()

    base = wid * EPWC
    bufs = ((ebuf0, sem0), (ebuf1, sem1))

    def issue(j, p):
        eb, se = bufs[p]
        pltpu.async_copy(ea_hbm.at[pl.ds(base + j * G, G)], eb, se)

    def drain_scatter(j, p):
        eb, se = bufs[p]
        pltpu.make_async_copy(ea_hbm.at[pl.ds(base + j * G, G)], eb,
                              se).wait()
        pltpu.sync_copy(eb, acc_sh.at[cidx_v.at[j]], add=True)

    issue(0, 0)

    @pl.loop(0, (NGC - 1) // 2)
    def _(t):
        j = 2 * t
        issue(j + 1, 1)
        drain_scatter(j, 0)
        issue(j + 2, 0)
        drain_scatter(j + 1, 1)

    drain_scatter(NGC - 1, 0)

    plsc.subcore_barrier()
    _acc_drain(c, s, acc_sh, acc_out)


def _sc_count_body(ones_hbm, col_hbm, znd_hbm, acc_out,
                   cidx_v, ebuf, acc_sh):
    c = lax.axis_index("c")
    s = lax.axis_index("s")
    wid = s * NC + c
    _acc_zero(s, znd_hbm, acc_sh)
    pltpu.sync_copy(ones_hbm, ebuf)
    pltpu.sync_copy(col_hbm.at[wid], cidx_v)
    plsc.subcore_barrier()

    @pl.loop(0, NG)
    def _(j):
        pltpu.sync_copy(ebuf, acc_sh.at[cidx_v.at[j]], add=True)

    plsc.subcore_barrier()
    _acc_drain(c, s, acc_sh, acc_out)


def _sc_scatter(ea_k, colk, zeros_nd):
    if "scatter" not in _sc_cache:
        _sc_cache["scatter"] = pl.kernel(
            _sc_scatter_data_body,
            out_type=jax.ShapeDtypeStruct((NC, N, D), jnp.float32),
            mesh=_sc_mesh(),
            scratch_types=[
                pltpu.VMEM((NGC, G), jnp.int32),
                pltpu.VMEM((G, D), jnp.float32),
                pltpu.VMEM((G, D), jnp.float32),
                pltpu.SemaphoreType.DMA,
                pltpu.SemaphoreType.DMA,
                pltpu.VMEM_SHARED((N, D), jnp.float32),
            ],
        )
    return _sc_cache["scatter"](ea_k, colk, zeros_nd)


def _sc_count(ones_gd, col3, zeros_nd):
    if "count" not in _sc_cache:
        _sc_cache["count"] = pl.kernel(
            _sc_count_body,
            out_type=jax.ShapeDtypeStruct((NC, N, D), jnp.float32),
            mesh=_sc_mesh(),
            scratch_types=[
                pltpu.VMEM((NG, G), jnp.int32),
                pltpu.VMEM((G, D), jnp.float32),
                pltpu.VMEM_SHARED((N, D), jnp.float32),
            ],
        )
    return _sc_cache["count"](ones_gd, col3, zeros_nd)


def _ln(h, g, b):
    mu = jnp.mean(h, axis=-1, keepdims=True)
    var = jnp.mean((h - mu) ** 2, axis=-1, keepdims=True)
    return (h - mu) * lax.rsqrt(var + 1e-5) * g + b


def _gelu(h):
    return 0.5 * h * (1.0 + lax.erf(h * 0.7071067811865476))


def _edge_mlp_body(xr_ref, xc_ref, ea_ref, w1a, w1b, w1c, b1, g1, bt1,
                   w2, b2, g2, bt2, out_ref):
    ea = ea_ref[...]
    h = (jnp.dot(xr_ref[...], w1a[...], preferred_element_type=jnp.float32)
         + jnp.dot(xc_ref[...], w1b[...], preferred_element_type=jnp.float32)
         + jnp.dot(ea.astype(jnp.bfloat16), w1c[...],
                   preferred_element_type=jnp.float32)
         + b1[...])
    h = _ln(h, g1[...], bt1[...])
    h = _gelu(h)
    h = jnp.dot(h.astype(jnp.bfloat16), w2[...],
                preferred_element_type=jnp.float32) + b2[...]
    h = _ln(h, g2[...], bt2[...])
    out_ref[...] = ea + h


_BE = 2560


def _edge_mlp(xr_k, xc_k, ea_arr, off, w1a, w1b, w1c, b1, g1, bt1,
              w2, b2, g2, bt2):
    def wspec(shape):
        return pl.BlockSpec(shape, lambda i: (0,) * len(shape))

    cb = pl.BlockSpec((_BE, D), lambda i: (i, 0))
    eb = pl.BlockSpec((_BE, D), lambda i: (i + off, 0))
    return pl.pallas_call(
        _edge_mlp_body,
        grid=(EK // _BE,),
        in_specs=[cb, cb, eb,
                  wspec((D, 2 * D)), wspec((D, 2 * D)), wspec((D, 2 * D)),
                  wspec((1, 2 * D)), wspec((1, 2 * D)), wspec((1, 2 * D)),
                  wspec((2 * D, D)), wspec((1, D)), wspec((1, D)),
                  wspec((1, D))],
        out_specs=cb,
        out_shape=jax.ShapeDtypeStruct((EK, D), jnp.float32),
    )(xr_k, xc_k, ea_arr, w1a, w1b, w1c, b1, g1, bt1, w2, b2, g2, bt2)


def _node_mlp_body(x_ref, p0, p1, p2, p3, p4, c_ref, w1a, w1b, b1, g1,
                   bt1, w2, b2, g2, bt2, out_ref):
    x = x_ref[...]
    cnt = (c_ref[0] + c_ref[1])[:, :1]
    acc = (p0[0] + p0[1] + p1[0] + p1[1] + p2[0] + p2[1]
           + p3[0] + p3[1] + p4[0] + p4[1])
    agg = acc / jnp.maximum(cnt, 1.0)
    h = (jnp.dot(x, w1a[...], preferred_element_type=jnp.float32)
         + jnp.dot(agg, w1b[...], preferred_element_type=jnp.float32)
         + b1[...])
    h = _ln(h, g1[...], bt1[...])
    h = _gelu(h)
    h = jnp.dot(h, w2[...], preferred_element_type=jnp.float32) + b2[...]
    h = _ln(h, g2[...], bt2[...])
    out_ref[...] = x + h


_BN = 2000


def _node_mlp(x, parts, cnt, w1a, w1b, b1, g1, bt1, w2, b2, g2, bt2):
    def wspec(shape):
        return pl.BlockSpec(shape, lambda i: (0,) * len(shape))

    nb = pl.BlockSpec((_BN, D), lambda i: (i, 0))
    pb = pl.BlockSpec((NC, _BN, D), lambda i: (0, i, 0))
    return pl.pallas_call(
        _node_mlp_body,
        grid=(N // _BN,),
        in_specs=[nb, pb, pb, pb, pb, pb, pb,
                  wspec((D, 2 * D)), wspec((D, 2 * D)),
                  wspec((1, 2 * D)), wspec((1, 2 * D)), wspec((1, 2 * D)),
                  wspec((2 * D, D)), wspec((1, D)), wspec((1, D)),
                  wspec((1, D))],
        out_specs=nb,
        out_shape=jax.ShapeDtypeStruct((N, D), jnp.float32),
    )(x, *parts, cnt, w1a, w1b, b1, g1, bt1, w2, b2, g2, bt2)


def kernel(x, edge_index, edge_attr, eW1, eb1, eg1, ebt1, eW2, eb2, eg2, ebt2,
           nW1, nb1, ng1, nbt1, nW2, nb2, ng2, nbt2):
    row4 = edge_index[0].reshape(K, NW, NGC, G)
    col4 = edge_index[1].reshape(K, NW, NGC, G)
    col3 = edge_index[1].reshape(NW, NG, G)
    zeros_nd = jnp.zeros((N, D), jnp.float32)
    ones_gd = jnp.ones((G, D), jnp.float32)

    xx = x
    ea_chunks = None
    cnt = None
    for i in range(L):
        bf = jnp.bfloat16
        ew = (eW1[i, :D].astype(bf), eW1[i, D:2 * D].astype(bf),
              eW1[i, 2 * D:].astype(bf),
              eb1[i][None, :], eg1[i][None, :], ebt1[i][None, :],
              eW2[i].astype(bf), eb2[i][None, :], eg2[i][None, :],
              ebt2[i][None, :])
        xb = xx.astype(bf)
        gathered = [_sc_gather(xb, row4[k], col4[k]) for k in range(K)]
        if i == 0:
            cnt = _sc_count(ones_gd, col3, zeros_nd)
        new_chunks, parts = [], []
        for k in range(K):
            xr_k, xc_k = gathered[k]
            if ea_chunks is None:
                ea_in, off = edge_attr, k * (EK // _BE)
            else:
                ea_in, off = ea_chunks[k], 0
            eak = _edge_mlp(xr_k, xc_k, ea_in, off, *ew)
            new_chunks.append(eak)
            parts.append(_sc_scatter(eak, col4[k], zeros_nd))
        ea_chunks = new_chunks
        xx = _node_mlp(
            xx, parts, cnt,
            nW1[i, :D], nW1[i, D:],
            nb1[i][None, :], ng1[i][None, :], nbt1[i][None, :],
            nW2[i], nb2[i][None, :], ng2[i][None, :], nbt2[i][None, :])
    return (xx, jnp.concatenate(ea_chunks, axis=0))


# speedup vs baseline: 3.0096x; 3.0096x over previous
import jax
import jax.numpy as jnp
from jax import lax
from jax.experimental import pallas as pl
from jax.experimental.pallas import tpu as pltpu
from jax.experimental.pallas import tpu_sc as plsc

L = 2
D = 128
N = 10000
E = 320000

NC = 2
NS = 16
NW = NC * NS
G = 80
K = 5
EK = E // K
EPWC = EK // NW
NGC = EPWC // G
EPW = E // NW
NG = EPW // G
RC = 80
NRC = N // RC

_sc_cache = {}


def _sc_mesh():
    return plsc.VectorSubcoreMesh(
        core_axis_name="c", subcore_axis_name="s",
        num_cores=NC, num_subcores=NS)


def _sc_gather_body(x_hbm, row_hbm, col_hbm, xr_hbm, xc_hbm,
                    ridx_v, cidx_v, rbuf0, rbuf1, cbuf0, cbuf1,
                    sem_r0, sem_r1, sem_c0, sem_c1):
    wid = lax.axis_index("s") * NC + lax.axis_index("c")
    pltpu.sync_copy(row_hbm.at[wid], ridx_v)
    pltpu.sync_copy(col_hbm.at[wid], cidx_v)
    base = wid * EPWC
    bufs = ((rbuf0, cbuf0, sem_r0, sem_c0), (rbuf1, cbuf1, sem_r1, sem_c1))

    def issue(j, p):
        rb, cb, sr, sc = bufs[p]
        pltpu.async_copy(x_hbm.at[ridx_v.at[j]], rb, sr)
        pltpu.async_copy(x_hbm.at[cidx_v.at[j]], cb, sc)

    def drain_write(j, p):
        rb, cb, sr, sc = bufs[p]
        pltpu.make_async_copy(x_hbm.at[ridx_v.at[j]], rb, sr).wait()
        pltpu.sync_copy(rb, xr_hbm.at[pl.ds(base + j * G, G)])
        pltpu.make_async_copy(x_hbm.at[cidx_v.at[j]], cb, sc).wait()
        pltpu.sync_copy(cb, xc_hbm.at[pl.ds(base + j * G, G)])

    issue(0, 0)

    @pl.loop(0, (NGC - 1) // 2)
    def _(t):
        j = 2 * t
        issue(j + 1, 1)
        drain_write(j, 0)
        issue(j + 2, 0)
        drain_write(j + 1, 1)

    drain_write(NGC - 1, 0)


def _sc_gather(x, rowk, colk):
    if "gather" not in _sc_cache:
        _sc_cache["gather"] = pl.kernel(
            _sc_gather_body,
            out_type=(
                jax.ShapeDtypeStruct((EK, D), jnp.float32),
                jax.ShapeDtypeStruct((EK, D), jnp.float32),
            ),
            mesh=_sc_mesh(),
            scratch_types=[
                pltpu.VMEM((NGC, G), jnp.int32),
                pltpu.VMEM((NGC, G), jnp.int32),
                pltpu.VMEM((G, D), jnp.float32),
                pltpu.VMEM((G, D), jnp.float32),
                pltpu.VMEM((G, D), jnp.float32),
                pltpu.VMEM((G, D), jnp.float32),
                pltpu.SemaphoreType.DMA,
                pltpu.SemaphoreType.DMA,
                pltpu.SemaphoreType.DMA,
                pltpu.SemaphoreType.DMA,
            ],
        )
    return _sc_cache["gather"](x, rowk, colk)


def _acc_zero(s, znd_hbm, acc_sh):
    for t in range((NRC + NS - 1) // NS):
        k = s + NS * t

        @pl.when(k < NRC)
        def _():
            pltpu.sync_copy(znd_hbm.at[pl.ds(k * RC, RC)],
                            acc_sh.at[pl.ds(k * RC, RC)])


def _acc_drain(c, s, acc_sh, acc_out):
    for t in range((NRC + NS - 1) // NS):
        k = s + NS * t

        @pl.when(k < NRC)
        def _():
            pltpu.sync_copy(acc_sh.at[pl.ds(k * RC, RC)],
                            acc_out.at[c, pl.ds(k * RC, RC)])


def _sc_scatter_data_body(ea_hbm, col_hbm, znd_hbm, acc_out,
                          cidx_v, ebuf0, ebuf1, sem0, sem1, acc_sh):
    c = lax.axis_index("c")
    s = lax.axis_index("s")
    wid = s * NC + c
    _acc_zero(s, znd_hbm, acc_sh)
    pltpu.sync_copy(col_hbm.at[wid], cidx_v)
    plsc.subcore_barrier()

    base = wid * EPWC
    bufs = ((ebuf0, sem0), (ebuf1, sem1))

    def issue(j, p):
        eb, se = bufs[p]
        pltpu.async_copy(ea_hbm.at[pl.ds(base + j * G, G)], eb, se)

    def drain_scatter(j, p):
        eb, se = bufs[p]
        pltpu.make_async_copy(ea_hbm.at[pl.ds(base + j * G, G)], eb,
                              se).wait()
        pltpu.sync_copy(eb, acc_sh.at[cidx_v.at[j]], add=True)

    issue(0, 0)

    @pl.loop(0, (NGC - 1) // 2)
    def _(t):
        j = 2 * t
        issue(j + 1, 1)
        drain_scatter(j, 0)
        issue(j + 2, 0)
        drain_scatter(j + 1, 1)

    drain_scatter(NGC - 1, 0)

    plsc.subcore_barrier()
    _acc_drain(c, s, acc_sh, acc_out)


def _sc_count_body(ones_hbm, col_hbm, znd_hbm, acc_out,
                   cidx_v, ebuf, acc_sh):
    c = lax.axis_index("c")
    s = lax.axis_index("s")
    wid = s * NC + c
    _acc_zero(s, znd_hbm, acc_sh)
    pltpu.sync_copy(ones_hbm, ebuf)
    pltpu.sync_copy(col_hbm.at[wid], cidx_v)
    plsc.subcore_barrier()

    @pl.loop(0, NG)
    def _(j):
        pltpu.sync_copy(ebuf, acc_sh.at[cidx_v.at[j]], add=True)

    plsc.subcore_barrier()
    _acc_drain(c, s, acc_sh, acc_out)


def _sc_scatter(ea_k, colk, zeros_nd):
    if "scatter" not in _sc_cache:
        _sc_cache["scatter"] = pl.kernel(
            _sc_scatter_data_body,
            out_type=jax.ShapeDtypeStruct((NC, N, D), jnp.float32),
            mesh=_sc_mesh(),
            scratch_types=[
                pltpu.VMEM((NGC, G), jnp.int32),
                pltpu.VMEM((G, D), jnp.float32),
                pltpu.VMEM((G, D), jnp.float32),
                pltpu.SemaphoreType.DMA,
                pltpu.SemaphoreType.DMA,
                pltpu.VMEM_SHARED((N, D), jnp.float32),
            ],
        )
    return _sc_cache["scatter"](ea_k, colk, zeros_nd)


def _sc_count(ones_gd, col3, zeros_nd):
    if "count" not in _sc_cache:
        _sc_cache["count"] = pl.kernel(
            _sc_count_body,
            out_type=jax.ShapeDtypeStruct((NC, N, D), jnp.float32),
            mesh=_sc_mesh(),
            scratch_types=[
                pltpu.VMEM((NG, G), jnp.int32),
                pltpu.VMEM((G, D), jnp.float32),
                pltpu.VMEM_SHARED((N, D), jnp.float32),
            ],
        )
    return _sc_cache["count"](ones_gd, col3, zeros_nd)


def _ln(h, g, b):
    mu = jnp.mean(h, axis=-1, keepdims=True)
    var = jnp.mean((h - mu) ** 2, axis=-1, keepdims=True)
    return (h - mu) * lax.rsqrt(var + 1e-5) * g + b


def _gelu(h):
    return 0.5 * h * (1.0 + lax.erf(h * 0.7071067811865476))


def _edge_mlp_body(xr_ref, xc_ref, ea_ref, w1a, w1b, w1c, b1, g1, bt1,
                   w2, b2, g2, bt2, out_ref):
    ea = ea_ref[...]
    h = (jnp.dot(xr_ref[...].astype(jnp.bfloat16), w1a[...],
                 preferred_element_type=jnp.float32)
         + jnp.dot(xc_ref[...].astype(jnp.bfloat16), w1b[...],
                   preferred_element_type=jnp.float32)
         + jnp.dot(ea.astype(jnp.bfloat16), w1c[...],
                   preferred_element_type=jnp.float32)
         + b1[...])
    h = _ln(h, g1[...], bt1[...])
    h = _gelu(h)
    h = jnp.dot(h.astype(jnp.bfloat16), w2[...],
                preferred_element_type=jnp.float32) + b2[...]
    h = _ln(h, g2[...], bt2[...])
    out_ref[...] = ea + h


_BE = 2560


def _edge_mlp(xr_k, xc_k, ea_arr, off, w1a, w1b, w1c, b1, g1, bt1,
              w2, b2, g2, bt2):
    def wspec(shape):
        return pl.BlockSpec(shape, lambda i: (0,) * len(shape))

    cb = pl.BlockSpec((_BE, D), lambda i: (i, 0))
    eb = pl.BlockSpec((_BE, D), lambda i: (i + off, 0))
    return pl.pallas_call(
        _edge_mlp_body,
        grid=(EK // _BE,),
        in_specs=[cb, cb, eb,
                  wspec((D, 2 * D)), wspec((D, 2 * D)), wspec((D, 2 * D)),
                  wspec((1, 2 * D)), wspec((1, 2 * D)), wspec((1, 2 * D)),
                  wspec((2 * D, D)), wspec((1, D)), wspec((1, D)),
                  wspec((1, D))],
        out_specs=cb,
        out_shape=jax.ShapeDtypeStruct((EK, D), jnp.float32),
    )(xr_k, xc_k, ea_arr, w1a, w1b, w1c, b1, g1, bt1, w2, b2, g2, bt2)


def _edge_mlp_body_full(xr_ref, xc_ref, ea_ref, fin_ref, w1a, w1b, w1c, b1,
                        g1, bt1, w2, b2, g2, bt2, out_ref, full_ref):
    ea = ea_ref[...]
    h = (jnp.dot(xr_ref[...].astype(jnp.bfloat16), w1a[...],
                 preferred_element_type=jnp.float32)
         + jnp.dot(xc_ref[...].astype(jnp.bfloat16), w1b[...],
                   preferred_element_type=jnp.float32)
         + jnp.dot(ea.astype(jnp.bfloat16), w1c[...],
                   preferred_element_type=jnp.float32)
         + b1[...])
    h = _ln(h, g1[...], bt1[...])
    h = _gelu(h)
    h = jnp.dot(h.astype(jnp.bfloat16), w2[...],
                preferred_element_type=jnp.float32) + b2[...]
    h = _ln(h, g2[...], bt2[...])
    res = ea + h
    out_ref[...] = res
    full_ref[...] = res


def _edge_mlp_full(xr_k, xc_k, ea_k, full_buf, off, w1a, w1b, w1c, b1, g1,
                   bt1, w2, b2, g2, bt2):
    def wspec(shape):
        return pl.BlockSpec(shape, lambda i: (0,) * len(shape))

    cb = pl.BlockSpec((_BE, D), lambda i: (i, 0))
    fb = pl.BlockSpec((_BE, D), lambda i: (i + off, 0))
    dummy = pl.BlockSpec((8, D), lambda i: (0, 0))
    return pl.pallas_call(
        _edge_mlp_body_full,
        grid=(EK // _BE,),
        in_specs=[cb, cb, cb, dummy,
                  wspec((D, 2 * D)), wspec((D, 2 * D)), wspec((D, 2 * D)),
                  wspec((1, 2 * D)), wspec((1, 2 * D)), wspec((1, 2 * D)),
                  wspec((2 * D, D)), wspec((1, D)), wspec((1, D)),
                  wspec((1, D))],
        out_specs=[cb, fb],
        out_shape=[jax.ShapeDtypeStruct((EK, D), jnp.float32),
                   jax.ShapeDtypeStruct((E, D), jnp.float32)],
        input_output_aliases={3: 1},
    )(xr_k, xc_k, ea_k, full_buf, w1a, w1b, w1c, b1, g1, bt1, w2, b2, g2, bt2)


def _node_mlp_body(x_ref, p0, p1, p2, p3, p4, c_ref, w1a, w1b, b1, g1,
                   bt1, w2, b2, g2, bt2, out_ref):
    x = x_ref[...]
    cnt = (c_ref[0] + c_ref[1])[:, :1]
    acc = (p0[0] + p0[1] + p1[0] + p1[1] + p2[0] + p2[1]
           + p3[0] + p3[1] + p4[0] + p4[1])
    agg = acc / jnp.maximum(cnt, 1.0)
    h = (jnp.dot(x, w1a[...], preferred_element_type=jnp.float32)
         + jnp.dot(agg, w1b[...], preferred_element_type=jnp.float32)
         + b1[...])
    h = _ln(h, g1[...], bt1[...])
    h = _gelu(h)
    h = jnp.dot(h, w2[...], preferred_element_type=jnp.float32) + b2[...]
    h = _ln(h, g2[...], bt2[...])
    out_ref[...] = x + h


_BN = 2000


def _node_mlp(x, parts, cnt, w1a, w1b, b1, g1, bt1, w2, b2, g2, bt2):
    def wspec(shape):
        return pl.BlockSpec(shape, lambda i: (0,) * len(shape))

    nb = pl.BlockSpec((_BN, D), lambda i: (i, 0))
    pb = pl.BlockSpec((NC, _BN, D), lambda i: (0, i, 0))
    return pl.pallas_call(
        _node_mlp_body,
        grid=(N // _BN,),
        in_specs=[nb, pb, pb, pb, pb, pb, pb,
                  wspec((D, 2 * D)), wspec((D, 2 * D)),
                  wspec((1, 2 * D)), wspec((1, 2 * D)), wspec((1, 2 * D)),
                  wspec((2 * D, D)), wspec((1, D)), wspec((1, D)),
                  wspec((1, D))],
        out_specs=nb,
        out_shape=jax.ShapeDtypeStruct((N, D), jnp.float32),
    )(x, *parts, cnt, w1a, w1b, b1, g1, bt1, w2, b2, g2, bt2)


def kernel(x, edge_index, edge_attr, eW1, eb1, eg1, ebt1, eW2, eb2, eg2, ebt2,
           nW1, nb1, ng1, nbt1, nW2, nb2, ng2, nbt2):
    row4 = edge_index[0].reshape(K, NW, NGC, G)
    col4 = edge_index[1].reshape(K, NW, NGC, G)
    col3 = edge_index[1].reshape(NW, NG, G)
    zeros_nd = jnp.zeros((N, D), jnp.float32)
    ones_gd = jnp.ones((G, D), jnp.float32)

    xx = x
    ea_chunks = None
    cnt = None
    for i in range(L):
        bf = jnp.bfloat16
        ew = (eW1[i, :D].astype(bf), eW1[i, D:2 * D].astype(bf),
              eW1[i, 2 * D:].astype(bf),
              eb1[i][None, :], eg1[i][None, :], ebt1[i][None, :],
              eW2[i].astype(bf), eb2[i][None, :], eg2[i][None, :],
              ebt2[i][None, :])
        gathered = [_sc_gather(xx, row4[k], col4[k]) for k in range(K)]
        if i == 0:
            cnt = _sc_count(ones_gd, col3, zeros_nd)
        if i == L - 1:
            ea_full = jnp.zeros((E, D), jnp.float32)
        new_chunks, parts = [], []
        for k in range(K):
            xr_k, xc_k = gathered[k]
            if ea_chunks is None:
                ea_in, off = edge_attr, k * (EK // _BE)
            else:
                ea_in, off = ea_chunks[k], 0
            if i == L - 1:
                eak, ea_full = _edge_mlp_full(
                    xr_k, xc_k, ea_in, ea_full, k * (EK // _BE), *ew)
            else:
                eak = _edge_mlp(xr_k, xc_k, ea_in, off, *ew)
            new_chunks.append(eak)
            parts.append(_sc_scatter(eak, col4[k], zeros_nd))
        ea_chunks = new_chunks
        xx = _node_mlp(
            xx, parts, cnt,
            nW1[i, :D], nW1[i, D:],
            nb1[i][None, :], ng1[i][None, :], nbt1[i][None, :],
            nW2[i], nb2[i][None, :], ng2[i][None, :], nbt2[i][None, :])
    return (xx, ea_full)


# speedup vs baseline: 3.0180x; 1.0028x over previous
import jax
import jax.numpy as jnp
from jax import lax
from jax.experimental import pallas as pl
from jax.experimental.pallas import tpu as pltpu
from jax.experimental.pallas import tpu_sc as plsc

L = 2
D = 128
N = 10000
E = 320000

NC = 2
NS = 16
NW = NC * NS
G = 80
K = 5
EK = E // K
EPWC = EK // NW
NGC = EPWC // G
EPW = E // NW
NG = EPW // G
RC = 80
NRC = N // RC

_sc_cache = {}


def _sc_mesh():
    return plsc.VectorSubcoreMesh(
        core_axis_name="c", subcore_axis_name="s",
        num_cores=NC, num_subcores=NS)


def _sc_gather_body(x_hbm, row_hbm, col_hbm, xr_hbm, xc_hbm,
                    ridx_v, cidx_v, rbuf0, rbuf1, cbuf0, cbuf1,
                    sem_r0, sem_r1, sem_c0, sem_c1):
    wid = lax.axis_index("s") * NC + lax.axis_index("c")
    pltpu.sync_copy(row_hbm.at[wid], ridx_v)
    pltpu.sync_copy(col_hbm.at[wid], cidx_v)
    base = wid * EPWC
    bufs = ((rbuf0, cbuf0, sem_r0, sem_c0), (rbuf1, cbuf1, sem_r1, sem_c1))

    def issue(j, p):
        rb, cb, sr, sc = bufs[p]
        pltpu.async_copy(x_hbm.at[ridx_v.at[j]], rb, sr)
        pltpu.async_copy(x_hbm.at[cidx_v.at[j]], cb, sc)

    def drain_write(j, p):
        rb, cb, sr, sc = bufs[p]
        pltpu.make_async_copy(x_hbm.at[ridx_v.at[j]], rb, sr).wait()
        pltpu.sync_copy(rb, xr_hbm.at[pl.ds(base + j * G, G)])
        pltpu.make_async_copy(x_hbm.at[cidx_v.at[j]], cb, sc).wait()
        pltpu.sync_copy(cb, xc_hbm.at[pl.ds(base + j * G, G)])

    issue(0, 0)

    @pl.loop(0, (NGC - 1) // 2)
    def _(t):
        j = 2 * t
        issue(j + 1, 1)
        drain_write(j, 0)
        issue(j + 2, 0)
        drain_write(j + 1, 1)

    drain_write(NGC - 1, 0)


def _sc_gather(x, rowk, colk):
    if "gather" not in _sc_cache:
        _sc_cache["gather"] = pl.kernel(
            _sc_gather_body,
            out_type=(
                jax.ShapeDtypeStruct((EK, D), jnp.float32),
                jax.ShapeDtypeStruct((EK, D), jnp.float32),
            ),
            mesh=_sc_mesh(),
            scratch_types=[
                pltpu.VMEM((NGC, G), jnp.int32),
                pltpu.VMEM((NGC, G), jnp.int32),
                pltpu.VMEM((G, D), jnp.float32),
                pltpu.VMEM((G, D), jnp.float32),
                pltpu.VMEM((G, D), jnp.float32),
                pltpu.VMEM((G, D), jnp.float32),
                pltpu.SemaphoreType.DMA,
                pltpu.SemaphoreType.DMA,
                pltpu.SemaphoreType.DMA,
                pltpu.SemaphoreType.DMA,
            ],
        )
    return _sc_cache["gather"](x, rowk, colk)


def _acc_zero(s, znd_hbm, acc_sh):
    for t in range((NRC + NS - 1) // NS):
        k = s + NS * t

        @pl.when(k < NRC)
        def _():
            pltpu.sync_copy(znd_hbm.at[pl.ds(k * RC, RC)],
                            acc_sh.at[pl.ds(k * RC, RC)])


def _acc_drain(c, s, acc_sh, acc_out):
    for t in range((NRC + NS - 1) // NS):
        k = s + NS * t

        @pl.when(k < NRC)
        def _():
            pltpu.sync_copy(acc_sh.at[pl.ds(k * RC, RC)],
                            acc_out.at[c, pl.ds(k * RC, RC)])


def _sc_scatter_data_body(ea_hbm, col_hbm, znd_hbm, acc_out,
                          cidx_v, ebuf0, ebuf1, sem0, sem1, acc_sh):
    c = lax.axis_index("c")
    s = lax.axis_index("s")
    wid = s * NC + c
    _acc_zero(s, znd_hbm, acc_sh)
    pltpu.sync_copy(col_hbm.at[wid], cidx_v)
    plsc.subcore_barrier()

    base = wid * EPWC
    bufs = ((ebuf0, sem0), (ebuf1, sem1))

    def issue(j, p):
        eb, se = bufs[p]
        pltpu.async_copy(ea_hbm.at[pl.ds(base + j * G, G)], eb, se)

    def drain_scatter(j, p):
        eb, se = bufs[p]
        pltpu.make_async_copy(ea_hbm.at[pl.ds(base + j * G, G)], eb,
                              se).wait()
        pltpu.sync_copy(eb, acc_sh.at[cidx_v.at[j]], add=True)

    issue(0, 0)

    @pl.loop(0, (NGC - 1) // 2)
    def _(t):
        j = 2 * t
        issue(j + 1, 1)
        drain_scatter(j, 0)
        issue(j + 2, 0)
        drain_scatter(j + 1, 1)

    drain_scatter(NGC - 1, 0)

    plsc.subcore_barrier()
    _acc_drain(c, s, acc_sh, acc_out)


def _sc_count_body(ones_hbm, col_hbm, znd_hbm, acc_out,
                   cidx_v, ebuf, acc_sh):
    c = lax.axis_index("c")
    s = lax.axis_index("s")
    wid = s * NC + c
    _acc_zero(s, znd_hbm, acc_sh)
    pltpu.sync_copy(ones_hbm, ebuf)
    pltpu.sync_copy(col_hbm.at[wid], cidx_v)
    plsc.subcore_barrier()

    @pl.loop(0, NG)
    def _(j):
        pltpu.sync_copy(ebuf, acc_sh.at[cidx_v.at[j]], add=True)

    plsc.subcore_barrier()
    _acc_drain(c, s, acc_sh, acc_out)


def _sc_scatter(ea_k, colk, zeros_nd):
    if "scatter" not in _sc_cache:
        _sc_cache["scatter"] = pl.kernel(
            _sc_scatter_data_body,
            out_type=jax.ShapeDtypeStruct((NC, N, D), jnp.float32),
            mesh=_sc_mesh(),
            scratch_types=[
                pltpu.VMEM((NGC, G), jnp.int32),
                pltpu.VMEM((G, D), jnp.float32),
                pltpu.VMEM((G, D), jnp.float32),
                pltpu.SemaphoreType.DMA,
                pltpu.SemaphoreType.DMA,
                pltpu.VMEM_SHARED((N, D), jnp.float32),
            ],
        )
    return _sc_cache["scatter"](ea_k, colk, zeros_nd)


def _sc_count(ones_gd, col3, zeros_nd):
    if "count" not in _sc_cache:
        _sc_cache["count"] = pl.kernel(
            _sc_count_body,
            out_type=jax.ShapeDtypeStruct((NC, N, D), jnp.float32),
            mesh=_sc_mesh(),
            scratch_types=[
                pltpu.VMEM((NG, G), jnp.int32),
                pltpu.VMEM((G, D), jnp.float32),
                pltpu.VMEM_SHARED((N, D), jnp.float32),
            ],
        )
    return _sc_cache["count"](ones_gd, col3, zeros_nd)


def _ln(h, g, b):
    mu = jnp.mean(h, axis=-1, keepdims=True)
    var = jnp.mean((h - mu) ** 2, axis=-1, keepdims=True)
    return (h - mu) * lax.rsqrt(var + 1e-5) * g + b


def _gelu(h):
    return 0.5 * h * (1.0 + lax.erf(h * 0.7071067811865476))


def _edge_mlp_body(xr_ref, xc_ref, ea_ref, w1a, w1b, w1c, b1, g1, bt1,
                   w2, b2, g2, bt2, out_ref):
    ea = ea_ref[...]
    h = (jnp.dot(xr_ref[...], w1a[...], preferred_element_type=jnp.float32)
         + jnp.dot(xc_ref[...], w1b[...], preferred_element_type=jnp.float32)
         + jnp.dot(ea, w1c[...], preferred_element_type=jnp.float32)
         + b1[...])
    h = _ln(h, g1[...], bt1[...])
    h = _gelu(h)
    h = jnp.dot(h, w2[...], preferred_element_type=jnp.float32) + b2[...]
    h = _ln(h, g2[...], bt2[...])
    out_ref[...] = ea + h


_BE = 2560


def _edge_mlp(xr_k, xc_k, ea_arr, off, w1a, w1b, w1c, b1, g1, bt1,
              w2, b2, g2, bt2):
    def wspec(shape):
        return pl.BlockSpec(shape, lambda i: (0,) * len(shape))

    cb = pl.BlockSpec((_BE, D), lambda i: (i, 0))
    eb = pl.BlockSpec((_BE, D), lambda i: (i + off, 0))
    return pl.pallas_call(
        _edge_mlp_body,
        grid=(EK // _BE,),
        in_specs=[cb, cb, eb,
                  wspec((D, 2 * D)), wspec((D, 2 * D)), wspec((D, 2 * D)),
                  wspec((1, 2 * D)), wspec((1, 2 * D)), wspec((1, 2 * D)),
                  wspec((2 * D, D)), wspec((1, D)), wspec((1, D)),
                  wspec((1, D))],
        out_specs=cb,
        out_shape=jax.ShapeDtypeStruct((EK, D), jnp.float32),
    )(xr_k, xc_k, ea_arr, w1a, w1b, w1c, b1, g1, bt1, w2, b2, g2, bt2)


def _edge_mlp_body_full(xr_ref, xc_ref, ea_ref, fin_ref, w1a, w1b, w1c, b1,
                        g1, bt1, w2, b2, g2, bt2, out_ref, full_ref):
    ea = ea_ref[...]
    h = (jnp.dot(xr_ref[...], w1a[...], preferred_element_type=jnp.float32)
         + jnp.dot(xc_ref[...], w1b[...], preferred_element_type=jnp.float32)
         + jnp.dot(ea, w1c[...], preferred_element_type=jnp.float32)
         + b1[...])
    h = _ln(h, g1[...], bt1[...])
    h = _gelu(h)
    h = jnp.dot(h, w2[...], preferred_element_type=jnp.float32) + b2[...]
    h = _ln(h, g2[...], bt2[...])
    res = ea + h
    out_ref[...] = res
    full_ref[...] = res


def _edge_mlp_full(xr_k, xc_k, ea_k, full_buf, off, w1a, w1b, w1c, b1, g1,
                   bt1, w2, b2, g2, bt2):
    def wspec(shape):
        return pl.BlockSpec(shape, lambda i: (0,) * len(shape))

    cb = pl.BlockSpec((_BE, D), lambda i: (i, 0))
    fb = pl.BlockSpec((_BE, D), lambda i: (i + off, 0))
    dummy = pl.BlockSpec((8, D), lambda i: (0, 0))
    return pl.pallas_call(
        _edge_mlp_body_full,
        grid=(EK // _BE,),
        in_specs=[cb, cb, cb, dummy,
                  wspec((D, 2 * D)), wspec((D, 2 * D)), wspec((D, 2 * D)),
                  wspec((1, 2 * D)), wspec((1, 2 * D)), wspec((1, 2 * D)),
                  wspec((2 * D, D)), wspec((1, D)), wspec((1, D)),
                  wspec((1, D))],
        out_specs=[cb, fb],
        out_shape=[jax.ShapeDtypeStruct((EK, D), jnp.float32),
                   jax.ShapeDtypeStruct((E, D), jnp.float32)],
        input_output_aliases={3: 1},
    )(xr_k, xc_k, ea_k, full_buf, w1a, w1b, w1c, b1, g1, bt1, w2, b2, g2, bt2)


def _node_mlp_body(x_ref, p0, p1, p2, p3, p4, c_ref, w1a, w1b, b1, g1,
                   bt1, w2, b2, g2, bt2, out_ref):
    x = x_ref[...]
    cnt = (c_ref[0] + c_ref[1])[:, :1]
    acc = (p0[0] + p0[1] + p1[0] + p1[1] + p2[0] + p2[1]
           + p3[0] + p3[1] + p4[0] + p4[1])
    agg = acc / jnp.maximum(cnt, 1.0)
    h = (jnp.dot(x, w1a[...], preferred_element_type=jnp.float32)
         + jnp.dot(agg, w1b[...], preferred_element_type=jnp.float32)
         + b1[...])
    h = _ln(h, g1[...], bt1[...])
    h = _gelu(h)
    h = jnp.dot(h, w2[...], preferred_element_type=jnp.float32) + b2[...]
    h = _ln(h, g2[...], bt2[...])
    out_ref[...] = x + h


_BN = 2000


def _node_mlp(x, parts, cnt, w1a, w1b, b1, g1, bt1, w2, b2, g2, bt2):
    def wspec(shape):
        return pl.BlockSpec(shape, lambda i: (0,) * len(shape))

    nb = pl.BlockSpec((_BN, D), lambda i: (i, 0))
    pb = pl.BlockSpec((NC, _BN, D), lambda i: (0, i, 0))
    return pl.pallas_call(
        _node_mlp_body,
        grid=(N // _BN,),
        in_specs=[nb, pb, pb, pb, pb, pb, pb,
                  wspec((D, 2 * D)), wspec((D, 2 * D)),
                  wspec((1, 2 * D)), wspec((1, 2 * D)), wspec((1, 2 * D)),
                  wspec((2 * D, D)), wspec((1, D)), wspec((1, D)),
                  wspec((1, D))],
        out_specs=nb,
        out_shape=jax.ShapeDtypeStruct((N, D), jnp.float32),
    )(x, *parts, cnt, w1a, w1b, b1, g1, bt1, w2, b2, g2, bt2)


def kernel(x, edge_index, edge_attr, eW1, eb1, eg1, ebt1, eW2, eb2, eg2, ebt2,
           nW1, nb1, ng1, nbt1, nW2, nb2, ng2, nbt2):
    row4 = edge_index[0].reshape(K, NW, NGC, G)
    col4 = edge_index[1].reshape(K, NW, NGC, G)
    col3 = edge_index[1].reshape(NW, NG, G)
    zeros_nd = jnp.zeros((N, D), jnp.float32)
    ones_gd = jnp.ones((G, D), jnp.float32)

    xx = x
    ea_chunks = None
    cnt = None
    for i in range(L):
        ew = (eW1[i, :D], eW1[i, D:2 * D], eW1[i, 2 * D:],
              eb1[i][None, :], eg1[i][None, :], ebt1[i][None, :],
              eW2[i], eb2[i][None, :], eg2[i][None, :], ebt2[i][None, :])
        gathered = [_sc_gather(xx, row4[k], col4[k]) for k in range(K)]
        if i == 0:
            cnt = _sc_count(ones_gd, col3, zeros_nd)
        if i == L - 1:
            ea_full = jnp.zeros((E, D), jnp.float32)
        new_chunks, parts = [], []
        for k in range(K):
            xr_k, xc_k = gathered[k]
            if ea_chunks is None:
                ea_in, off = edge_attr, k * (EK // _BE)
            else:
                ea_in, off = ea_chunks[k], 0
            if i == L - 1:
                eak, ea_full = _edge_mlp_full(
                    xr_k, xc_k, ea_in, ea_full, k * (EK // _BE), *ew)
            else:
                eak = _edge_mlp(xr_k, xc_k, ea_in, off, *ew)
            new_chunks.append(eak)
            parts.append(_sc_scatter(eak, col4[k], zeros_nd))
        ea_chunks = new_chunks
        xx = _node_mlp(
            xx, parts, cnt,
            nW1[i, :D], nW1[i, D:],
            nb1[i][None, :], ng1[i][None, :], nbt1[i][None, :],
            nW2[i], nb2[i][None, :], ng2[i][None, :], nbt2[i][None, :])
    return (xx, ea_full)


# speedup vs baseline: 3.3826x; 1.1208x over previous
import jax
import jax.numpy as jnp
from jax import lax
from jax.experimental import pallas as pl
from jax.experimental.pallas import tpu as pltpu
from jax.experimental.pallas import tpu_sc as plsc

L = 2
D = 128
N = 10000
E = 320000

NC = 2
NS = 16
NW = NC * NS
G = 80
K = 5
EK = E // K
EPWC = EK // NW
NGC = EPWC // G
EPW = E // NW
NG = EPW // G
RC = 80
NRC = N // RC

_sc_cache = {}


def _sc_mesh():
    return plsc.VectorSubcoreMesh(
        core_axis_name="c", subcore_axis_name="s",
        num_cores=NC, num_subcores=NS)


def _sc_gather_body(x_hbm, row_hbm, col_hbm, xr_hbm, xc_hbm,
                    ridx_v, cidx_v, rbuf0, rbuf1, cbuf0, cbuf1,
                    sem_r0, sem_r1, sem_c0, sem_c1):
    wid = lax.axis_index("s") * NC + lax.axis_index("c")
    pltpu.sync_copy(row_hbm.at[wid], ridx_v)
    pltpu.sync_copy(col_hbm.at[wid], cidx_v)
    base = wid * EPWC
    bufs = ((rbuf0, cbuf0, sem_r0, sem_c0), (rbuf1, cbuf1, sem_r1, sem_c1))

    def issue(j, p):
        rb, cb, sr, sc = bufs[p]
        pltpu.async_copy(x_hbm.at[ridx_v.at[j]], rb, sr)
        pltpu.async_copy(x_hbm.at[cidx_v.at[j]], cb, sc)

    def drain_write(j, p):
        rb, cb, sr, sc = bufs[p]
        pltpu.make_async_copy(x_hbm.at[ridx_v.at[j]], rb, sr).wait()
        pltpu.sync_copy(rb, xr_hbm.at[pl.ds(base + j * G, G)])
        pltpu.make_async_copy(x_hbm.at[cidx_v.at[j]], cb, sc).wait()
        pltpu.sync_copy(cb, xc_hbm.at[pl.ds(base + j * G, G)])

    issue(0, 0)

    @pl.loop(0, (NGC - 1) // 2)
    def _(t):
        j = 2 * t
        issue(j + 1, 1)
        drain_write(j, 0)
        issue(j + 2, 0)
        drain_write(j + 1, 1)

    drain_write(NGC - 1, 0)


def _sc_gather(x, rowk, colk):
    if "gather" not in _sc_cache:
        _sc_cache["gather"] = pl.kernel(
            _sc_gather_body,
            out_type=(
                jax.ShapeDtypeStruct((EK, D), jnp.float32),
                jax.ShapeDtypeStruct((EK, D), jnp.float32),
            ),
            mesh=_sc_mesh(),
            scratch_types=[
                pltpu.VMEM((NGC, G), jnp.int32),
                pltpu.VMEM((NGC, G), jnp.int32),
                pltpu.VMEM((G, D), jnp.float32),
                pltpu.VMEM((G, D), jnp.float32),
                pltpu.VMEM((G, D), jnp.float32),
                pltpu.VMEM((G, D), jnp.float32),
                pltpu.SemaphoreType.DMA,
                pltpu.SemaphoreType.DMA,
                pltpu.SemaphoreType.DMA,
                pltpu.SemaphoreType.DMA,
            ],
        )
    return _sc_cache["gather"](x, rowk, colk)


def _acc_zero(s, znd_hbm, acc_sh):
    for t in range((NRC + NS - 1) // NS):
        k = s + NS * t

        @pl.when(k < NRC)
        def _():
            pltpu.sync_copy(znd_hbm.at[pl.ds(k * RC, RC)],
                            acc_sh.at[pl.ds(k * RC, RC)])


def _acc_drain(c, s, acc_sh, acc_out):
    for t in range((NRC + NS - 1) // NS):
        k = s + NS * t

        @pl.when(k < NRC)
        def _():
            pltpu.sync_copy(acc_sh.at[pl.ds(k * RC, RC)],
                            acc_out.at[c, pl.ds(k * RC, RC)])


def _sc_scatter_data_body(ea_hbm, col_hbm, znd_hbm, acc_out,
                          cidx_v, ebuf0, ebuf1, sem0, sem1, acc_sh):
    c = lax.axis_index("c")
    s = lax.axis_index("s")
    wid = s * NC + c
    _acc_zero(s, znd_hbm, acc_sh)
    pltpu.sync_copy(col_hbm.at[wid], cidx_v)
    plsc.subcore_barrier()

    base = wid * EPWC
    bufs = ((ebuf0, sem0), (ebuf1, sem1))

    def issue(j, p):
        eb, se = bufs[p]
        pltpu.async_copy(ea_hbm.at[pl.ds(base + j * G, G)], eb, se)

    def drain_scatter(j, p):
        eb, se = bufs[p]
        pltpu.make_async_copy(ea_hbm.at[pl.ds(base + j * G, G)], eb,
                              se).wait()
        pltpu.sync_copy(eb, acc_sh.at[cidx_v.at[j]], add=True)

    issue(0, 0)

    @pl.loop(0, (NGC - 1) // 2)
    def _(t):
        j = 2 * t
        issue(j + 1, 1)
        drain_scatter(j, 0)
        issue(j + 2, 0)
        drain_scatter(j + 1, 1)

    drain_scatter(NGC - 1, 0)

    plsc.subcore_barrier()
    _acc_drain(c, s, acc_sh, acc_out)


def _sc_count_body(ones_hbm, col_hbm, znd_hbm, acc_out,
                   cidx_v, ebuf, acc_sh):
    c = lax.axis_index("c")
    s = lax.axis_index("s")
    wid = s * NC + c
    _acc_zero(s, znd_hbm, acc_sh)
    pltpu.sync_copy(ones_hbm, ebuf)
    pltpu.sync_copy(col_hbm.at[wid], cidx_v)
    plsc.subcore_barrier()

    @pl.loop(0, NG)
    def _(j):
        pltpu.sync_copy(ebuf, acc_sh.at[cidx_v.at[j]], add=True)

    plsc.subcore_barrier()
    _acc_drain(c, s, acc_sh, acc_out)


def _sc_scatter(ea_k, colk, zeros_nd):
    if "scatter" not in _sc_cache:
        _sc_cache["scatter"] = pl.kernel(
            _sc_scatter_data_body,
            out_type=jax.ShapeDtypeStruct((NC, N, D), jnp.float32),
            mesh=_sc_mesh(),
            scratch_types=[
                pltpu.VMEM((NGC, G), jnp.int32),
                pltpu.VMEM((G, D), jnp.float32),
                pltpu.VMEM((G, D), jnp.float32),
                pltpu.SemaphoreType.DMA,
                pltpu.SemaphoreType.DMA,
                pltpu.VMEM_SHARED((N, D), jnp.float32),
            ],
        )
    return _sc_cache["scatter"](ea_k, colk, zeros_nd)


def _sc_count(ones_gd, col3, zeros_nd):
    if "count" not in _sc_cache:
        _sc_cache["count"] = pl.kernel(
            _sc_count_body,
            out_type=jax.ShapeDtypeStruct((NC, N, D), jnp.float32),
            mesh=_sc_mesh(),
            scratch_types=[
                pltpu.VMEM((NG, G), jnp.int32),
                pltpu.VMEM((G, D), jnp.float32),
                pltpu.VMEM_SHARED((N, D), jnp.float32),
            ],
        )
    return _sc_cache["count"](ones_gd, col3, zeros_nd)


def _ln(h, g, b):
    mu = jnp.mean(h, axis=-1, keepdims=True)
    var = jnp.mean((h - mu) ** 2, axis=-1, keepdims=True)
    return (h - mu) * lax.rsqrt(var + 1e-5) * g + b


def _gelu(h):
    return 0.5 * h * (1.0 + lax.erf(h * 0.7071067811865476))


def _edge_mlp_body(xr_ref, xc_ref, ea_ref, w1a, w1b, w1c, b1, g1, bt1,
                   w2, b2, g2, bt2, out_ref):
    ea = ea_ref[...]
    h = (jnp.dot(xr_ref[...], w1a[...], preferred_element_type=jnp.float32)
         + jnp.dot(xc_ref[...], w1b[...], preferred_element_type=jnp.float32)
         + jnp.dot(ea, w1c[...], preferred_element_type=jnp.float32)
         + b1[...])
    h = _ln(h, g1[...], bt1[...])
    h = _gelu(h)
    h = jnp.dot(h, w2[...], preferred_element_type=jnp.float32) + b2[...]
    h = _ln(h, g2[...], bt2[...])
    out_ref[...] = ea + h


_BE = 2560


def _edge_mlp(xr_k, xc_k, ea_arr, off, w1a, w1b, w1c, b1, g1, bt1,
              w2, b2, g2, bt2):
    def wspec(shape):
        return pl.BlockSpec(shape, lambda i: (0,) * len(shape))

    cb = pl.BlockSpec((_BE, D), lambda i: (i, 0))
    eb = pl.BlockSpec((_BE, D), lambda i: (i + off, 0))
    return pl.pallas_call(
        _edge_mlp_body,
        grid=(EK // _BE,),
        in_specs=[cb, cb, eb,
                  wspec((D, 2 * D)), wspec((D, 2 * D)), wspec((D, 2 * D)),
                  wspec((1, 2 * D)), wspec((1, 2 * D)), wspec((1, 2 * D)),
                  wspec((2 * D, D)), wspec((1, D)), wspec((1, D)),
                  wspec((1, D))],
        out_specs=cb,
        out_shape=jax.ShapeDtypeStruct((EK, D), jnp.float32),
    )(xr_k, xc_k, ea_arr, w1a, w1b, w1c, b1, g1, bt1, w2, b2, g2, bt2)


def _edge_mlp_body_full(xr_ref, xc_ref, ea_ref, fin_ref, w1a, w1b, w1c, b1,
                        g1, bt1, w2, b2, g2, bt2, out_ref, full_ref):
    ea = ea_ref[...]
    h = (jnp.dot(xr_ref[...], w1a[...], preferred_element_type=jnp.float32)
         + jnp.dot(xc_ref[...], w1b[...], preferred_element_type=jnp.float32)
         + jnp.dot(ea, w1c[...], preferred_element_type=jnp.float32)
         + b1[...])
    h = _ln(h, g1[...], bt1[...])
    h = _gelu(h)
    h = jnp.dot(h, w2[...], preferred_element_type=jnp.float32) + b2[...]
    h = _ln(h, g2[...], bt2[...])
    res = ea + h
    out_ref[...] = res
    full_ref[...] = res


def _edge_mlp_full(xr_k, xc_k, ea_k, full_buf, off, w1a, w1b, w1c, b1, g1,
                   bt1, w2, b2, g2, bt2):
    def wspec(shape):
        return pl.BlockSpec(shape, lambda i: (0,) * len(shape))

    cb = pl.BlockSpec((_BE, D), lambda i: (i, 0))
    fb = pl.BlockSpec((_BE, D), lambda i: (i + off, 0))
    dummy = pl.BlockSpec((8, D), lambda i: (0, 0))
    return pl.pallas_call(
        _edge_mlp_body_full,
        grid=(EK // _BE,),
        in_specs=[cb, cb, cb, dummy,
                  wspec((D, 2 * D)), wspec((D, 2 * D)), wspec((D, 2 * D)),
                  wspec((1, 2 * D)), wspec((1, 2 * D)), wspec((1, 2 * D)),
                  wspec((2 * D, D)), wspec((1, D)), wspec((1, D)),
                  wspec((1, D))],
        out_specs=[cb, fb],
        out_shape=[jax.ShapeDtypeStruct((EK, D), jnp.float32),
                   jax.ShapeDtypeStruct((E, D), jnp.float32)],
        input_output_aliases={3: 1},
    )(xr_k, xc_k, ea_k, full_buf, w1a, w1b, w1c, b1, g1, bt1, w2, b2, g2, bt2)


def _node_mlp_body(x_ref, p0, p1, p2, p3, p4, c_ref, w1a, w1b, b1, g1,
                   bt1, w2, b2, g2, bt2, out_ref):
    x = x_ref[...]
    cnt = (c_ref[0] + c_ref[1])[:, :1]
    acc = (p0[0] + p0[1] + p1[0] + p1[1] + p2[0] + p2[1]
           + p3[0] + p3[1] + p4[0] + p4[1])
    agg = acc / jnp.maximum(cnt, 1.0)
    h = (jnp.dot(x, w1a[...], preferred_element_type=jnp.float32)
         + jnp.dot(agg, w1b[...], preferred_element_type=jnp.float32)
         + b1[...])
    h = _ln(h, g1[...], bt1[...])
    h = _gelu(h)
    h = jnp.dot(h, w2[...], preferred_element_type=jnp.float32) + b2[...]
    h = _ln(h, g2[...], bt2[...])
    out_ref[...] = x + h


_BN = 2000


def _node_mlp(x, parts, cnt, w1a, w1b, b1, g1, bt1, w2, b2, g2, bt2):
    def wspec(shape):
        return pl.BlockSpec(shape, lambda i: (0,) * len(shape))

    nb = pl.BlockSpec((_BN, D), lambda i: (i, 0))
    pb = pl.BlockSpec((NC, _BN, D), lambda i: (0, i, 0))
    return pl.pallas_call(
        _node_mlp_body,
        grid=(N // _BN,),
        in_specs=[nb, pb, pb, pb, pb, pb, pb,
                  wspec((D, 2 * D)), wspec((D, 2 * D)),
                  wspec((1, 2 * D)), wspec((1, 2 * D)), wspec((1, 2 * D)),
                  wspec((2 * D, D)), wspec((1, D)), wspec((1, D)),
                  wspec((1, D))],
        out_specs=nb,
        out_shape=jax.ShapeDtypeStruct((N, D), jnp.float32),
    )(x, *parts, cnt, w1a, w1b, b1, g1, bt1, w2, b2, g2, bt2)


def kernel(x, edge_index, edge_attr, eW1, eb1, eg1, ebt1, eW2, eb2, eg2, ebt2,
           nW1, nb1, ng1, nbt1, nW2, nb2, ng2, nbt2):
    row4 = edge_index[0].reshape(K, NW, NGC, G)
    col4 = edge_index[1].reshape(K, NW, NGC, G)
    col3 = edge_index[1].reshape(NW, NG, G)
    zeros_nd = jnp.zeros((N, D), jnp.float32)
    ones_gd = jnp.ones((G, D), jnp.float32)

    xx = x
    ea_chunks = None
    cnt = None
    for i in range(L):
        ew = (eW1[i, :D], eW1[i, D:2 * D], eW1[i, 2 * D:],
              eb1[i][None, :], eg1[i][None, :], ebt1[i][None, :],
              eW2[i], eb2[i][None, :], eg2[i][None, :], ebt2[i][None, :])
        gathered = [_sc_gather(xx, row4[k], col4[k]) for k in range(K)]
        if i == 0:
            cnt = _sc_count(ones_gd, col3, zeros_nd)
        new_chunks, parts = [], []
        for k in range(K):
            xr_k, xc_k = gathered[k]
            if ea_chunks is None:
                ea_in, off = edge_attr, k * (EK // _BE)
            else:
                ea_in, off = ea_chunks[k], 0
            eak = _edge_mlp(xr_k, xc_k, ea_in, off, *ew)
            new_chunks.append(eak)
            parts.append(_sc_scatter(eak, col4[k], zeros_nd))
        ea_chunks = new_chunks
        xx = _node_mlp(
            xx, parts, cnt,
            nW1[i, :D], nW1[i, D:],
            nb1[i][None, :], ng1[i][None, :], nbt1[i][None, :],
            nW2[i], nb2[i][None, :], ng2[i][None, :], nbt2[i][None, :])
    return (xx, jnp.concatenate(ea_chunks, axis=0))


# speedup vs baseline: 3.3923x; 1.0029x over previous
import jax
import jax.numpy as jnp
from jax import lax
from jax.experimental import pallas as pl
from jax.experimental.pallas import tpu as pltpu
from jax.experimental.pallas import tpu_sc as plsc

L = 2
D = 128
N = 10000
E = 320000

NC = 2
NS = 16
NW = NC * NS
G = 80
K = 5
EK = E // K
EPWC = EK // NW
NGC = EPWC // G
EPW = E // NW
NG = EPW // G
RC = 80
NRC = N // RC

_sc_cache = {}


def _sc_mesh():
    return plsc.VectorSubcoreMesh(
        core_axis_name="c", subcore_axis_name="s",
        num_cores=NC, num_subcores=NS)


def _sc_gather_body(x_hbm, row_hbm, col_hbm, xr_hbm, xc_hbm,
                    ridx_v, cidx_v, rbuf0, rbuf1, cbuf0, cbuf1,
                    sem_r0, sem_r1, sem_c0, sem_c1):
    wid = lax.axis_index("s") * NC + lax.axis_index("c")
    pltpu.sync_copy(row_hbm.at[wid], ridx_v)
    pltpu.sync_copy(col_hbm.at[wid], cidx_v)
    base = wid * EPWC
    bufs = ((rbuf0, cbuf0, sem_r0, sem_c0), (rbuf1, cbuf1, sem_r1, sem_c1))

    def issue(j, p):
        rb, cb, sr, sc = bufs[p]
        pltpu.async_copy(x_hbm.at[ridx_v.at[j]], rb, sr)
        pltpu.async_copy(x_hbm.at[cidx_v.at[j]], cb, sc)

    def drain_write(j, p):
        rb, cb, sr, sc = bufs[p]
        pltpu.make_async_copy(x_hbm.at[ridx_v.at[j]], rb, sr).wait()
        pltpu.sync_copy(rb, xr_hbm.at[pl.ds(base + j * G, G)])
        pltpu.make_async_copy(x_hbm.at[cidx_v.at[j]], cb, sc).wait()
        pltpu.sync_copy(cb, xc_hbm.at[pl.ds(base + j * G, G)])

    issue(0, 0)

    @pl.loop(0, (NGC - 1) // 2)
    def _(t):
        j = 2 * t
        issue(j + 1, 1)
        drain_write(j, 0)
        issue(j + 2, 0)
        drain_write(j + 1, 1)

    drain_write(NGC - 1, 0)


def _sc_gather(x, rowk, colk):
    if "gather" not in _sc_cache:
        _sc_cache["gather"] = pl.kernel(
            _sc_gather_body,
            out_type=(
                jax.ShapeDtypeStruct((EK, D), jnp.float32),
                jax.ShapeDtypeStruct((EK, D), jnp.float32),
            ),
            mesh=_sc_mesh(),
            scratch_types=[
                pltpu.VMEM((NGC, G), jnp.int32),
                pltpu.VMEM((NGC, G), jnp.int32),
                pltpu.VMEM((G, D), jnp.float32),
                pltpu.VMEM((G, D), jnp.float32),
                pltpu.VMEM((G, D), jnp.float32),
                pltpu.VMEM((G, D), jnp.float32),
                pltpu.SemaphoreType.DMA,
                pltpu.SemaphoreType.DMA,
                pltpu.SemaphoreType.DMA,
                pltpu.SemaphoreType.DMA,
            ],
        )
    return _sc_cache["gather"](x, rowk, colk)


def _acc_zero(s, znd_hbm, acc_sh):
    for t in range((NRC + NS - 1) // NS):
        k = s + NS * t

        @pl.when(k < NRC)
        def _():
            pltpu.sync_copy(znd_hbm.at[pl.ds(k * RC, RC)],
                            acc_sh.at[pl.ds(k * RC, RC)])


def _acc_drain(c, s, acc_sh, acc_out):
    for t in range((NRC + NS - 1) // NS):
        k = s + NS * t

        @pl.when(k < NRC)
        def _():
            pltpu.sync_copy(acc_sh.at[pl.ds(k * RC, RC)],
                            acc_out.at[c, pl.ds(k * RC, RC)])


def _sc_scatter_data_body(ea_hbm, col_hbm, znd_hbm, acc_out,
                          cidx_v, ebuf0, ebuf1, sem0, sem1, acc_sh):
    c = lax.axis_index("c")
    s = lax.axis_index("s")
    wid = s * NC + c
    _acc_zero(s, znd_hbm, acc_sh)
    pltpu.sync_copy(col_hbm.at[wid], cidx_v)
    plsc.subcore_barrier()

    base = wid * EPWC
    bufs = ((ebuf0, sem0), (ebuf1, sem1))

    def issue(j, p):
        eb, se = bufs[p]
        pltpu.async_copy(ea_hbm.at[pl.ds(base + j * G, G)], eb, se)

    def drain_scatter(j, p):
        eb, se = bufs[p]
        pltpu.make_async_copy(ea_hbm.at[pl.ds(base + j * G, G)], eb,
                              se).wait()
        pltpu.sync_copy(eb, acc_sh.at[cidx_v.at[j]], add=True)

    issue(0, 0)

    @pl.loop(0, (NGC - 1) // 2)
    def _(t):
        j = 2 * t
        issue(j + 1, 1)
        drain_scatter(j, 0)
        issue(j + 2, 0)
        drain_scatter(j + 1, 1)

    drain_scatter(NGC - 1, 0)

    plsc.subcore_barrier()
    _acc_drain(c, s, acc_sh, acc_out)


def _sc_count_body(ones_hbm, col_hbm, znd_hbm, acc_out,
                   cidx_v, ebuf, acc_sh):
    c = lax.axis_index("c")
    s = lax.axis_index("s")
    wid = s * NC + c
    _acc_zero(s, znd_hbm, acc_sh)
    pltpu.sync_copy(ones_hbm, ebuf)
    pltpu.sync_copy(col_hbm.at[wid], cidx_v)
    plsc.subcore_barrier()

    @pl.loop(0, NG)
    def _(j):
        pltpu.sync_copy(ebuf, acc_sh.at[cidx_v.at[j]], add=True)

    plsc.subcore_barrier()
    _acc_drain(c, s, acc_sh, acc_out)


def _sc_scatter(ea_k, colk, zeros_nd):
    if "scatter" not in _sc_cache:
        _sc_cache["scatter"] = pl.kernel(
            _sc_scatter_data_body,
            out_type=jax.ShapeDtypeStruct((NC, N, D), jnp.float32),
            mesh=_sc_mesh(),
            scratch_types=[
                pltpu.VMEM((NGC, G), jnp.int32),
                pltpu.VMEM((G, D), jnp.float32),
                pltpu.VMEM((G, D), jnp.float32),
                pltpu.SemaphoreType.DMA,
                pltpu.SemaphoreType.DMA,
                pltpu.VMEM_SHARED((N, D), jnp.float32),
            ],
        )
    return _sc_cache["scatter"](ea_k, colk, zeros_nd)


def _sc_count(ones_gd, col3, zeros_nd):
    if "count" not in _sc_cache:
        _sc_cache["count"] = pl.kernel(
            _sc_count_body,
            out_type=jax.ShapeDtypeStruct((NC, N, D), jnp.float32),
            mesh=_sc_mesh(),
            scratch_types=[
                pltpu.VMEM((NG, G), jnp.int32),
                pltpu.VMEM((G, D), jnp.float32),
                pltpu.VMEM_SHARED((N, D), jnp.float32),
            ],
        )
    return _sc_cache["count"](ones_gd, col3, zeros_nd)


def _ln(h, g, b):
    mu = jnp.mean(h, axis=-1, keepdims=True)
    var = jnp.mean((h - mu) ** 2, axis=-1, keepdims=True)
    return (h - mu) * lax.rsqrt(var + 1e-5) * g + b


def _gelu(h):
    return 0.5 * h * (1.0 + lax.erf(h * 0.7071067811865476))


def _edge_mlp_body(xr_ref, xc_ref, ea_ref, w1a, w1b, w1c, b1, g1, bt1,
                   w2, b2, g2, bt2, out_ref):
    ea = ea_ref[...]
    h = (jnp.dot(xr_ref[...], w1a[...], preferred_element_type=jnp.float32)
         + jnp.dot(xc_ref[...], w1b[...], preferred_element_type=jnp.float32)
         + jnp.dot(ea, w1c[...], preferred_element_type=jnp.float32)
         + b1[...])
    h = _ln(h, g1[...], bt1[...])
    h = _gelu(h)
    h = jnp.dot(h, w2[...], preferred_element_type=jnp.float32) + b2[...]
    h = _ln(h, g2[...], bt2[...])
    out_ref[...] = ea + h


_BE = 3200


def _edge_mlp(xr_k, xc_k, ea_arr, off, w1a, w1b, w1c, b1, g1, bt1,
              w2, b2, g2, bt2):
    def wspec(shape):
        return pl.BlockSpec(shape, lambda i: (0,) * len(shape))

    cb = pl.BlockSpec((_BE, D), lambda i: (i, 0))
    eb = pl.BlockSpec((_BE, D), lambda i: (i + off, 0))
    return pl.pallas_call(
        _edge_mlp_body,
        grid=(EK // _BE,),
        in_specs=[cb, cb, eb,
                  wspec((D, 2 * D)), wspec((D, 2 * D)), wspec((D, 2 * D)),
                  wspec((1, 2 * D)), wspec((1, 2 * D)), wspec((1, 2 * D)),
                  wspec((2 * D, D)), wspec((1, D)), wspec((1, D)),
                  wspec((1, D))],
        out_specs=cb,
        out_shape=jax.ShapeDtypeStruct((EK, D), jnp.float32),
    )(xr_k, xc_k, ea_arr, w1a, w1b, w1c, b1, g1, bt1, w2, b2, g2, bt2)


def _edge_mlp_body_full(xr_ref, xc_ref, ea_ref, fin_ref, w1a, w1b, w1c, b1,
                        g1, bt1, w2, b2, g2, bt2, out_ref, full_ref):
    ea = ea_ref[...]
    h = (jnp.dot(xr_ref[...], w1a[...], preferred_element_type=jnp.float32)
         + jnp.dot(xc_ref[...], w1b[...], preferred_element_type=jnp.float32)
         + jnp.dot(ea, w1c[...], preferred_element_type=jnp.float32)
         + b1[...])
    h = _ln(h, g1[...], bt1[...])
    h = _gelu(h)
    h = jnp.dot(h, w2[...], preferred_element_type=jnp.float32) + b2[...]
    h = _ln(h, g2[...], bt2[...])
    res = ea + h
    out_ref[...] = res
    full_ref[...] = res


def _edge_mlp_full(xr_k, xc_k, ea_k, full_buf, off, w1a, w1b, w1c, b1, g1,
                   bt1, w2, b2, g2, bt2):
    def wspec(shape):
        return pl.BlockSpec(shape, lambda i: (0,) * len(shape))

    cb = pl.BlockSpec((_BE, D), lambda i: (i, 0))
    fb = pl.BlockSpec((_BE, D), lambda i: (i + off, 0))
    dummy = pl.BlockSpec((8, D), lambda i: (0, 0))
    return pl.pallas_call(
        _edge_mlp_body_full,
        grid=(EK // _BE,),
        in_specs=[cb, cb, cb, dummy,
                  wspec((D, 2 * D)), wspec((D, 2 * D)), wspec((D, 2 * D)),
                  wspec((1, 2 * D)), wspec((1, 2 * D)), wspec((1, 2 * D)),
                  wspec((2 * D, D)), wspec((1, D)), wspec((1, D)),
                  wspec((1, D))],
        out_specs=[cb, fb],
        out_shape=[jax.ShapeDtypeStruct((EK, D), jnp.float32),
                   jax.ShapeDtypeStruct((E, D), jnp.float32)],
        input_output_aliases={3: 1},
    )(xr_k, xc_k, ea_k, full_buf, w1a, w1b, w1c, b1, g1, bt1, w2, b2, g2, bt2)


def _node_mlp_body(x_ref, p0, p1, p2, p3, p4, c_ref, w1a, w1b, b1, g1,
                   bt1, w2, b2, g2, bt2, out_ref):
    x = x_ref[...]
    cnt = (c_ref[0] + c_ref[1])[:, :1]
    acc = (p0[0] + p0[1] + p1[0] + p1[1] + p2[0] + p2[1]
           + p3[0] + p3[1] + p4[0] + p4[1])
    agg = acc / jnp.maximum(cnt, 1.0)
    h = (jnp.dot(x, w1a[...], preferred_element_type=jnp.float32)
         + jnp.dot(agg, w1b[...], preferred_element_type=jnp.float32)
         + b1[...])
    h = _ln(h, g1[...], bt1[...])
    h = _gelu(h)
    h = jnp.dot(h, w2[...], preferred_element_type=jnp.float32) + b2[...]
    h = _ln(h, g2[...], bt2[...])
    out_ref[...] = x + h


_BN = 2000


def _node_mlp(x, parts, cnt, w1a, w1b, b1, g1, bt1, w2, b2, g2, bt2):
    def wspec(shape):
        return pl.BlockSpec(shape, lambda i: (0,) * len(shape))

    nb = pl.BlockSpec((_BN, D), lambda i: (i, 0))
    pb = pl.BlockSpec((NC, _BN, D), lambda i: (0, i, 0))
    return pl.pallas_call(
        _node_mlp_body,
        grid=(N // _BN,),
        in_specs=[nb, pb, pb, pb, pb, pb, pb,
                  wspec((D, 2 * D)), wspec((D, 2 * D)),
                  wspec((1, 2 * D)), wspec((1, 2 * D)), wspec((1, 2 * D)),
                  wspec((2 * D, D)), wspec((1, D)), wspec((1, D)),
                  wspec((1, D))],
        out_specs=nb,
        out_shape=jax.ShapeDtypeStruct((N, D), jnp.float32),
    )(x, *parts, cnt, w1a, w1b, b1, g1, bt1, w2, b2, g2, bt2)


def kernel(x, edge_index, edge_attr, eW1, eb1, eg1, ebt1, eW2, eb2, eg2, ebt2,
           nW1, nb1, ng1, nbt1, nW2, nb2, ng2, nbt2):
    row4 = edge_index[0].reshape(K, NW, NGC, G)
    col4 = edge_index[1].reshape(K, NW, NGC, G)
    col3 = edge_index[1].reshape(NW, NG, G)
    zeros_nd = jnp.zeros((N, D), jnp.float32)
    ones_gd = jnp.ones((G, D), jnp.float32)

    xx = x
    ea_chunks = None
    cnt = None
    for i in range(L):
        ew = (eW1[i, :D], eW1[i, D:2 * D], eW1[i, 2 * D:],
              eb1[i][None, :], eg1[i][None, :], ebt1[i][None, :],
              eW2[i], eb2[i][None, :], eg2[i][None, :], ebt2[i][None, :])
        if i == 0:
            cnt = _sc_count(ones_gd, col3, zeros_nd)
        gathered = [_sc_gather(xx, row4[k], col4[k]) for k in range(K)]
        new_chunks, parts = [], []
        for k in range(K):
            xr_k, xc_k = gathered[k]
            if ea_chunks is None:
                ea_in, off = edge_attr, k * (EK // _BE)
            else:
                ea_in, off = ea_chunks[k], 0
            eak = _edge_mlp(xr_k, xc_k, ea_in, off, *ew)
            new_chunks.append(eak)
            parts.append(_sc_scatter(eak, col4[k], zeros_nd))
        ea_chunks = new_chunks
        xx = _node_mlp(
            xx, parts, cnt,
            nW1[i, :D], nW1[i, D:],
            nb1[i][None, :], ng1[i][None, :], nbt1[i][None, :],
            nW2[i], nb2[i][None, :], ng2[i][None, :], nbt2[i][None, :])
    return (xx, jnp.concatenate(ea_chunks, axis=0))


# speedup vs baseline: 3.4934x; 1.0298x over previous
import jax
import jax.numpy as jnp
from jax import lax
from jax.experimental import pallas as pl
from jax.experimental.pallas import tpu as pltpu
from jax.experimental.pallas import tpu_sc as plsc

L = 2
D = 128
N = 10000
E = 320000

NC = 2
NS = 16
NW = NC * NS
G = 80
K = 5
EK = E // K
EPWC = EK // NW
NGC = EPWC // G
EPW = E // NW
NG = EPW // G
RC = 80
NRC = N // RC

_sc_cache = {}


def _sc_mesh():
    return plsc.VectorSubcoreMesh(
        core_axis_name="c", subcore_axis_name="s",
        num_cores=NC, num_subcores=NS)


def _sc_gather_body(x_hbm, row_hbm, col_hbm, xr_hbm, xc_hbm,
                    ridx_v, cidx_v, rbuf0, rbuf1, cbuf0, cbuf1,
                    sem_r0, sem_r1, sem_c0, sem_c1):
    wid = lax.axis_index("s") * NC + lax.axis_index("c")
    pltpu.sync_copy(row_hbm.at[wid], ridx_v)
    pltpu.sync_copy(col_hbm.at[wid], cidx_v)
    base = wid * EPWC
    bufs = ((rbuf0, cbuf0, sem_r0, sem_c0), (rbuf1, cbuf1, sem_r1, sem_c1))

    def issue(j, p):
        rb, cb, sr, sc = bufs[p]
        pltpu.async_copy(x_hbm.at[ridx_v.at[j]], rb, sr)
        pltpu.async_copy(x_hbm.at[cidx_v.at[j]], cb, sc)

    def drain_write(j, p):
        rb, cb, sr, sc = bufs[p]
        pltpu.make_async_copy(x_hbm.at[ridx_v.at[j]], rb, sr).wait()
        pltpu.sync_copy(rb, xr_hbm.at[pl.ds(base + j * G, G)])
        pltpu.make_async_copy(x_hbm.at[cidx_v.at[j]], cb, sc).wait()
        pltpu.sync_copy(cb, xc_hbm.at[pl.ds(base + j * G, G)])

    issue(0, 0)

    @pl.loop(0, (NGC - 1) // 2)
    def _(t):
        j = 2 * t
        issue(j + 1, 1)
        drain_write(j, 0)
        issue(j + 2, 0)
        drain_write(j + 1, 1)

    drain_write(NGC - 1, 0)


def _sc_gather(x, rowk, colk):
    if "gather" not in _sc_cache:
        _sc_cache["gather"] = pl.kernel(
            _sc_gather_body,
            out_type=(
                jax.ShapeDtypeStruct((EK, D), jnp.float32),
                jax.ShapeDtypeStruct((EK, D), jnp.float32),
            ),
            mesh=_sc_mesh(),
            scratch_types=[
                pltpu.VMEM((NGC, G), jnp.int32),
                pltpu.VMEM((NGC, G), jnp.int32),
                pltpu.VMEM((G, D), jnp.float32),
                pltpu.VMEM((G, D), jnp.float32),
                pltpu.VMEM((G, D), jnp.float32),
                pltpu.VMEM((G, D), jnp.float32),
                pltpu.SemaphoreType.DMA,
                pltpu.SemaphoreType.DMA,
                pltpu.SemaphoreType.DMA,
                pltpu.SemaphoreType.DMA,
            ],
        )
    return _sc_cache["gather"](x, rowk, colk)


def _fill(buf, value):
    v = jnp.full((16,), value, jnp.float32)
    for r in range(RC):
        for q in range(D // 16):
            buf[r, pl.ds(q * 16, 16)] = v


def _acc_zero(s, zbuf, acc_sh):
    _fill(zbuf, 0.0)
    for t in range((NRC + NS - 1) // NS):
        k = s + NS * t

        @pl.when(k < NRC)
        def _():
            pltpu.sync_copy(zbuf, acc_sh.at[pl.ds(k * RC, RC)])


def _acc_drain(c, s, acc_sh, acc_out):
    for t in range((NRC + NS - 1) // NS):
        k = s + NS * t

        @pl.when(k < NRC)
        def _():
            pltpu.sync_copy(acc_sh.at[pl.ds(k * RC, RC)],
                            acc_out.at[c, pl.ds(k * RC, RC)])


def _sc_scatter_data_body(ea_hbm, col_hbm, acc_out,
                          cidx_v, ebuf0, ebuf1, sem0, sem1, acc_sh):
    c = lax.axis_index("c")
    s = lax.axis_index("s")
    wid = s * NC + c
    _acc_zero(s, ebuf0, acc_sh)
    pltpu.sync_copy(col_hbm.at[wid], cidx_v)
    plsc.subcore_barrier()

    base = wid * EPWC
    bufs = ((ebuf0, sem0), (ebuf1, sem1))

    def issue(j, p):
        eb, se = bufs[p]
        pltpu.async_copy(ea_hbm.at[pl.ds(base + j * G, G)], eb, se)

    def drain_scatter(j, p):
        eb, se = bufs[p]
        pltpu.make_async_copy(ea_hbm.at[pl.ds(base + j * G, G)], eb,
                              se).wait()
        pltpu.sync_copy(eb, acc_sh.at[cidx_v.at[j]], add=True)

    issue(0, 0)

    @pl.loop(0, (NGC - 1) // 2)
    def _(t):
        j = 2 * t
        issue(j + 1, 1)
        drain_scatter(j, 0)
        issue(j + 2, 0)
        drain_scatter(j + 1, 1)

    drain_scatter(NGC - 1, 0)

    plsc.subcore_barrier()
    _acc_drain(c, s, acc_sh, acc_out)


def _sc_count_body(col_hbm, acc_out, cidx_v, ebuf, acc_sh):
    c = lax.axis_index("c")
    s = lax.axis_index("s")
    wid = s * NC + c
    _acc_zero(s, ebuf, acc_sh)
    _fill(ebuf, 1.0)
    pltpu.sync_copy(col_hbm.at[wid], cidx_v)
    plsc.subcore_barrier()

    @pl.loop(0, NG)
    def _(j):
        pltpu.sync_copy(ebuf, acc_sh.at[cidx_v.at[j]], add=True)

    plsc.subcore_barrier()
    _acc_drain(c, s, acc_sh, acc_out)


def _sc_scatter(ea_k, colk):
    if "scatter" not in _sc_cache:
        _sc_cache["scatter"] = pl.kernel(
            _sc_scatter_data_body,
            out_type=jax.ShapeDtypeStruct((NC, N, D), jnp.float32),
            mesh=_sc_mesh(),
            scratch_types=[
                pltpu.VMEM((NGC, G), jnp.int32),
                pltpu.VMEM((G, D), jnp.float32),
                pltpu.VMEM((G, D), jnp.float32),
                pltpu.SemaphoreType.DMA,
                pltpu.SemaphoreType.DMA,
                pltpu.VMEM_SHARED((N, D), jnp.float32),
            ],
        )
    return _sc_cache["scatter"](ea_k, colk)


def _sc_count(col3):
    if "count" not in _sc_cache:
        _sc_cache["count"] = pl.kernel(
            _sc_count_body,
            out_type=jax.ShapeDtypeStruct((NC, N, D), jnp.float32),
            mesh=_sc_mesh(),
            scratch_types=[
                pltpu.VMEM((NG, G), jnp.int32),
                pltpu.VMEM((G, D), jnp.float32),
                pltpu.VMEM_SHARED((N, D), jnp.float32),
            ],
        )
    return _sc_cache["count"](col3)


def _ln(h, g, b):
    mu = jnp.mean(h, axis=-1, keepdims=True)
    var = jnp.mean((h - mu) ** 2, axis=-1, keepdims=True)
    return (h - mu) * lax.rsqrt(var + 1e-5) * g + b


def _gelu(h):
    return 0.5 * h * (1.0 + lax.erf(h * 0.7071067811865476))


def _edge_mlp_body(xr_ref, xc_ref, ea_ref, w1a, w1b, w1c, b1, g1, bt1,
                   w2, b2, g2, bt2, out_ref):
    ea = ea_ref[...]
    h = (jnp.dot(xr_ref[...], w1a[...], preferred_element_type=jnp.float32)
         + jnp.dot(xc_ref[...], w1b[...], preferred_element_type=jnp.float32)
         + jnp.dot(ea, w1c[...], preferred_element_type=jnp.float32)
         + b1[...])
    h = _ln(h, g1[...], bt1[...])
    h = _gelu(h)
    h = jnp.dot(h, w2[...], preferred_element_type=jnp.float32) + b2[...]
    h = _ln(h, g2[...], bt2[...])
    out_ref[...] = ea + h


_BE = 3200


def _edge_mlp(xr_k, xc_k, ea_arr, off, w1a, w1b, w1c, b1, g1, bt1,
              w2, b2, g2, bt2):
    def wspec(shape):
        return pl.BlockSpec(shape, lambda i: (0,) * len(shape))

    cb = pl.BlockSpec((_BE, D), lambda i: (i, 0))
    eb = pl.BlockSpec((_BE, D), lambda i: (i + off, 0))
    return pl.pallas_call(
        _edge_mlp_body,
        grid=(EK // _BE,),
        in_specs=[cb, cb, eb,
                  wspec((D, 2 * D)), wspec((D, 2 * D)), wspec((D, 2 * D)),
                  wspec((1, 2 * D)), wspec((1, 2 * D)), wspec((1, 2 * D)),
                  wspec((2 * D, D)), wspec((1, D)), wspec((1, D)),
                  wspec((1, D))],
        out_specs=cb,
        out_shape=jax.ShapeDtypeStruct((EK, D), jnp.float32),
    )(xr_k, xc_k, ea_arr, w1a, w1b, w1c, b1, g1, bt1, w2, b2, g2, bt2)


def _edge_mlp_body_full(xr_ref, xc_ref, ea_ref, fin_ref, w1a, w1b, w1c, b1,
                        g1, bt1, w2, b2, g2, bt2, out_ref, full_ref):
    ea = ea_ref[...]
    h = (jnp.dot(xr_ref[...], w1a[...], preferred_element_type=jnp.float32)
         + jnp.dot(xc_ref[...], w1b[...], preferred_element_type=jnp.float32)
         + jnp.dot(ea, w1c[...], preferred_element_type=jnp.float32)
         + b1[...])
    h = _ln(h, g1[...], bt1[...])
    h = _gelu(h)
    h = jnp.dot(h, w2[...], preferred_element_type=jnp.float32) + b2[...]
    h = _ln(h, g2[...], bt2[...])
    res = ea + h
    out_ref[...] = res
    full_ref[...] = res


def _edge_mlp_full(xr_k, xc_k, ea_k, full_buf, off, w1a, w1b, w1c, b1, g1,
                   bt1, w2, b2, g2, bt2):
    def wspec(shape):
        return pl.BlockSpec(shape, lambda i: (0,) * len(shape))

    cb = pl.BlockSpec((_BE, D), lambda i: (i, 0))
    fb = pl.BlockSpec((_BE, D), lambda i: (i + off, 0))
    dummy = pl.BlockSpec((8, D), lambda i: (0, 0))
    return pl.pallas_call(
        _edge_mlp_body_full,
        grid=(EK // _BE,),
        in_specs=[cb, cb, cb, dummy,
                  wspec((D, 2 * D)), wspec((D, 2 * D)), wspec((D, 2 * D)),
                  wspec((1, 2 * D)), wspec((1, 2 * D)), wspec((1, 2 * D)),
                  wspec((2 * D, D)), wspec((1, D)), wspec((1, D)),
                  wspec((1, D))],
        out_specs=[cb, fb],
        out_shape=[jax.ShapeDtypeStruct((EK, D), jnp.float32),
                   jax.ShapeDtypeStruct((E, D), jnp.float32)],
        input_output_aliases={3: 1},
    )(xr_k, xc_k, ea_k, full_buf, w1a, w1b, w1c, b1, g1, bt1, w2, b2, g2, bt2)


def _node_mlp_body(x_ref, p0, p1, p2, p3, p4, c_ref, w1a, w1b, b1, g1,
                   bt1, w2, b2, g2, bt2, out_ref):
    x = x_ref[...]
    cnt = (c_ref[0] + c_ref[1])[:, :1]
    acc = (p0[0] + p0[1] + p1[0] + p1[1] + p2[0] + p2[1]
           + p3[0] + p3[1] + p4[0] + p4[1])
    agg = acc / jnp.maximum(cnt, 1.0)
    h = (jnp.dot(x, w1a[...], preferred_element_type=jnp.float32)
         + jnp.dot(agg, w1b[...], preferred_element_type=jnp.float32)
         + b1[...])
    h = _ln(h, g1[...], bt1[...])
    h = _gelu(h)
    h = jnp.dot(h, w2[...], preferred_element_type=jnp.float32) + b2[...]
    h = _ln(h, g2[...], bt2[...])
    out_ref[...] = x + h


_BN = 2000


def _node_mlp(x, parts, cnt, w1a, w1b, b1, g1, bt1, w2, b2, g2, bt2):
    def wspec(shape):
        return pl.BlockSpec(shape, lambda i: (0,) * len(shape))

    nb = pl.BlockSpec((_BN, D), lambda i: (i, 0))
    pb = pl.BlockSpec((NC, _BN, D), lambda i: (0, i, 0))
    return pl.pallas_call(
        _node_mlp_body,
        grid=(N // _BN,),
        in_specs=[nb, pb, pb, pb, pb, pb, pb,
                  wspec((D, 2 * D)), wspec((D, 2 * D)),
                  wspec((1, 2 * D)), wspec((1, 2 * D)), wspec((1, 2 * D)),
                  wspec((2 * D, D)), wspec((1, D)), wspec((1, D)),
                  wspec((1, D))],
        out_specs=nb,
        out_shape=jax.ShapeDtypeStruct((N, D), jnp.float32),
    )(x, *parts, cnt, w1a, w1b, b1, g1, bt1, w2, b2, g2, bt2)


def kernel(x, edge_index, edge_attr, eW1, eb1, eg1, ebt1, eW2, eb2, eg2, ebt2,
           nW1, nb1, ng1, nbt1, nW2, nb2, ng2, nbt2):
    row4 = edge_index[0].reshape(K, NW, NGC, G)
    col4 = edge_index[1].reshape(K, NW, NGC, G)
    col3 = edge_index[1].reshape(NW, NG, G)
    xx = x
    ea_chunks = None
    cnt = None
    for i in range(L):
        ew = (eW1[i, :D], eW1[i, D:2 * D], eW1[i, 2 * D:],
              eb1[i][None, :], eg1[i][None, :], ebt1[i][None, :],
              eW2[i], eb2[i][None, :], eg2[i][None, :], ebt2[i][None, :])
        if i == 0:
            cnt = _sc_count(col3)
        gathered = [_sc_gather(xx, row4[k], col4[k]) for k in range(K)]
        new_chunks, parts = [], []
        for k in range(K):
            xr_k, xc_k = gathered[k]
            if ea_chunks is None:
                ea_in, off = edge_attr, k * (EK // _BE)
            else:
                ea_in, off = ea_chunks[k], 0
            eak = _edge_mlp(xr_k, xc_k, ea_in, off, *ew)
            new_chunks.append(eak)
            parts.append(_sc_scatter(eak, col4[k]))
        ea_chunks = new_chunks
        xx = _node_mlp(
            xx, parts, cnt,
            nW1[i, :D], nW1[i, D:],
            nb1[i][None, :], ng1[i][None, :], nbt1[i][None, :],
            nW2[i], nb2[i][None, :], ng2[i][None, :], nbt2[i][None, :])
    return (xx, jnp.concatenate(ea_chunks, axis=0))
